# Initial kernel scaffold; baseline (speedup 1.0000x reference)
import jax
import jax.numpy as jnp
from jax.experimental import pallas as pl


def kernel(x, y, SoS):
    raise NotImplementedError("write your pallas kernel here")


# speedup vs baseline: 659.3336x; 659.3336x over previous
import functools

import jax
import jax.numpy as jnp
from jax import lax
from jax.experimental import pallas as pl
from jax.experimental.pallas import tpu as pltpu
from jax.experimental.pallas import tpu_sc as plsc

N_POINTS = 3600
N_INT = 2000
R_BODY = 10.0
V0 = 1540.0
X0 = -12.0
DX = 1.6
Y0 = -12.0
DY = 1.6

L = 16
NC = 2
NS = 16
NW = NC * NS
N_GROUPS = N_POINTS // L
MAX_G_PER_W = -(-N_GROUPS // NW)
UNROLL = 8
N_ROUNDS = N_INT // UNROLL


def _sc_body(bx_hbm, by_hbm, sc_hbm, ax_hbm, ay_hbm, tbl_hbm, wf_hbm,
             tbl_v, bx_v, by_v, sc_v, ax_v, ay_v, out_v):
    wid = lax.axis_index("s") * NC + lax.axis_index("c")
    pltpu.sync_copy(tbl_hbm, tbl_v)
    pltpu.sync_copy(ax_hbm, ax_v)
    pltpu.sync_copy(ay_hbm, ay_v)
    axv = ax_v[...]
    ayv = ay_v[...]

    for rr in range(MAX_G_PER_W):
        g = wid + rr * NW

        @pl.when(g < N_GROUPS)
        def _():
            base = g * L
            pltpu.sync_copy(bx_hbm.at[pl.ds(base, L)], bx_v)
            pltpu.sync_copy(by_hbm.at[pl.ds(base, L)], by_v)
            pltpu.sync_copy(sc_hbm.at[pl.ds(base, L)], sc_v)
            bx = bx_v[...]
            by = by_v[...]
            bxu = bx * float(UNROLL)
            byu = by * float(UNROLL)

            def gather_at(xf, yf):
                xi = jnp.minimum(xf.astype(jnp.int32), 15)
                yi = yf.astype(jnp.int32)
                row = (-yi) & 15
                return plsc.load_gather(tbl_v, [(row << 4) + xi])

            zero = jnp.zeros((L,), jnp.float32)
            xfs = tuple(axv + bx * float(k) for k in range(UNROLL))
            yfs = tuple(ayv + by * float(k) for k in range(UNROLL))
            accs = (zero,) * UNROLL

            def body(_, carry):
                xfs, yfs, accs = carry
                nxf, nyf, nacc = [], [], []
                for k in range(UNROLL):
                    nacc.append(accs[k] + gather_at(xfs[k], yfs[k]))
                    nxf.append(xfs[k] + bxu)
                    nyf.append(yfs[k] + byu)
                return tuple(nxf), tuple(nyf), tuple(nacc)

            _, _, accs = lax.fori_loop(0, N_ROUNDS, body, (xfs, yfs, accs))
            acc = functools.reduce(jnp.add, accs)

            v_first = gather_at(axv, ayv)
            v_last = gather_at(axv + bx * float(N_INT - 1),
                               ayv + by * float(N_INT - 1))
            out_v[...] = sc_v[...] * (2.0 * acc - v_first - v_last)
            pltpu.sync_copy(out_v, wf_hbm.at[pl.ds(base, L)])


@functools.cache
def _sc_integrate():
    return pl.kernel(
        _sc_body,
        out_type=jax.ShapeDtypeStruct((N_POINTS,), jnp.float32),
        mesh=plsc.VectorSubcoreMesh(core_axis_name="c", subcore_axis_name="s"),
        compiler_params=pltpu.CompilerParams(needs_layout_passes=False),
        scratch_types=[
            pltpu.VMEM((256,), jnp.float32),
            pltpu.VMEM((L,), jnp.float32),
            pltpu.VMEM((L,), jnp.float32),
            pltpu.VMEM((L,), jnp.float32),
            pltpu.VMEM((L,), jnp.float32),
            pltpu.VMEM((L,), jnp.float32),
            pltpu.VMEM((L,), jnp.float32),
        ],
    )


def kernel(x, y, SoS):
    thetas = jnp.linspace(0.0, 2.0 * jnp.pi, N_POINTS, dtype=jnp.float32)
    r = jnp.sqrt(x ** 2 + y ** 2)
    phi = jnp.arctan2(x, y)
    t = thetas - phi
    chord = jnp.sqrt(R_BODY ** 2 - (r * jnp.sin(t)) ** 2)
    l_inside = chord + r * jnp.cos(t)
    l_outside = 2.0 * chord * (jnp.cos(phi - thetas) >= 0.0).astype(jnp.float32)
    l = jnp.where(r[0] < R_BODY, l_inside, l_outside)

    ax = jnp.full((L,), (x[0] - X0) / DX + 0.5, dtype=jnp.float32)
    ay = jnp.full((L,), (y[0] - Y0) / DY + 0.5, dtype=jnp.float32)
    bx = -(l * jnp.sin(thetas)) / jnp.float32(DX * (N_INT - 1))
    by = -(l * jnp.cos(thetas)) / jnp.float32(DY * (N_INT - 1))
    scale = l / jnp.float32(2 * (N_INT - 1))
    tbl = (1.0 - V0 / SoS).astype(jnp.float32).reshape(-1)

    wf = _sc_integrate()(bx, by, scale, ax, ay, tbl)
    return thetas, wf


# speedup vs baseline: 696.4275x; 1.0563x over previous
import functools

import jax
import jax.numpy as jnp
from jax import lax
from jax.experimental import pallas as pl
from jax.experimental.pallas import tpu as pltpu
from jax.experimental.pallas import tpu_sc as plsc

N_POINTS = 3600
N_INT = 2000
R_BODY = 10.0
V0 = 1540.0
X0 = -12.0
DX = 1.6
Y0 = -12.0
DY = 1.6

L = 16
NC = 2
NS = 16
NW = NC * NS
N_GROUPS = N_POINTS // L
MAX_G_PER_W = -(-N_GROUPS // NW)
UNROLL = 8
N_ROUNDS = N_INT // UNROLL


def _sc_body(bx_hbm, by_hbm, sc_hbm, ax_hbm, ay_hbm, tbl_hbm, wf_hbm,
             tbl_v, bx_v, by_v, sc_v, ax_v, ay_v, out_v):
    wid = lax.axis_index("s") * NC + lax.axis_index("c")
    pltpu.sync_copy(tbl_hbm, tbl_v)
    pltpu.sync_copy(ax_hbm, ax_v)
    pltpu.sync_copy(ay_hbm, ay_v)
    axv = ax_v[...]
    ayv = ay_v[...]

    for rr in range(MAX_G_PER_W):
        g = wid + rr * NW

        @pl.when(g < N_GROUPS)
        def _():
            base = g * L
            pltpu.sync_copy(bx_hbm.at[pl.ds(base, L)], bx_v)
            pltpu.sync_copy(by_hbm.at[pl.ds(base, L)], by_v)
            pltpu.sync_copy(sc_hbm.at[pl.ds(base, L)], sc_v)
            bx = bx_v[...]
            by = by_v[...]
            bxu = bx * float(UNROLL)
            byu = by * float(UNROLL)

            def gather_at(xf, yf):
                xi = xf.astype(jnp.int32)
                row = yf.astype(jnp.int32) & 15
                return plsc.load_gather(tbl_v, [(row << 4) + xi])

            zero = jnp.zeros((L,), jnp.float32)
            xfs = tuple(axv + bx * float(k) for k in range(UNROLL))
            yfs = tuple(ayv + by * float(k) for k in range(UNROLL))
            accs = (zero,) * UNROLL

            def body(_, carry):
                xfs, yfs, accs = carry
                nxf, nyf, nacc = [], [], []
                for k in range(UNROLL):
                    nacc.append(accs[k] + gather_at(xfs[k], yfs[k]))
                    nxf.append(xfs[k] + bxu)
                    nyf.append(yfs[k] + byu)
                return tuple(nxf), tuple(nyf), tuple(nacc)

            _, _, accs = lax.fori_loop(0, N_ROUNDS, body, (xfs, yfs, accs))
            acc = functools.reduce(jnp.add, accs)

            v_first = gather_at(axv, ayv)
            v_last = gather_at(axv + bx * float(N_INT - 1),
                               ayv + by * float(N_INT - 1))
            out_v[...] = sc_v[...] * (2.0 * acc - v_first - v_last)
            pltpu.sync_copy(out_v, wf_hbm.at[pl.ds(base, L)])


@functools.cache
def _sc_integrate():
    return pl.kernel(
        _sc_body,
        out_type=jax.ShapeDtypeStruct((N_POINTS,), jnp.float32),
        mesh=plsc.VectorSubcoreMesh(core_axis_name="c", subcore_axis_name="s"),
        compiler_params=pltpu.CompilerParams(needs_layout_passes=False),
        scratch_types=[
            pltpu.VMEM((256,), jnp.float32),
            pltpu.VMEM((L,), jnp.float32),
            pltpu.VMEM((L,), jnp.float32),
            pltpu.VMEM((L,), jnp.float32),
            pltpu.VMEM((L,), jnp.float32),
            pltpu.VMEM((L,), jnp.float32),
            pltpu.VMEM((L,), jnp.float32),
        ],
    )


def kernel(x, y, SoS):
    thetas = jnp.linspace(0.0, 2.0 * jnp.pi, N_POINTS, dtype=jnp.float32)
    r = jnp.sqrt(x ** 2 + y ** 2)
    phi = jnp.arctan2(x, y)
    t = thetas - phi
    chord = jnp.sqrt(R_BODY ** 2 - (r * jnp.sin(t)) ** 2)
    l_inside = chord + r * jnp.cos(t)
    l_outside = 2.0 * chord * (jnp.cos(phi - thetas) >= 0.0).astype(jnp.float32)
    l = jnp.where(r[0] < R_BODY, l_inside, l_outside)

    ax = jnp.full((L,), (x[0] - X0) / DX + 0.5, dtype=jnp.float32)
    ay = jnp.full((L,), -((y[0] - Y0) / DY + 0.5), dtype=jnp.float32)
    bx = -(l * jnp.sin(thetas)) / jnp.float32(DX * (N_INT - 1))
    by = (l * jnp.cos(thetas)) / jnp.float32(DY * (N_INT - 1))
    scale = l / jnp.float32(2 * (N_INT - 1))
    tbl = (1.0 - V0 / SoS).astype(jnp.float32).reshape(-1)

    wf = _sc_integrate()(bx, by, scale, ax, ay, tbl)
    return thetas, wf


# speedup vs baseline: 697.8721x; 1.0021x over previous
import functools

import jax
import jax.numpy as jnp
from jax import lax
from jax.experimental import pallas as pl
from jax.experimental.pallas import tpu as pltpu
from jax.experimental.pallas import tpu_sc as plsc

N_POINTS = 3600
N_INT = 2000
R_BODY = 10.0
V0 = 1540.0
X0 = -12.0
DX = 1.6
Y0 = -12.0
DY = 1.6

L = 16
NC = 2
NS = 16
NW = NC * NS
N_GROUPS = N_POINTS // L
MAX_G_PER_W = -(-N_GROUPS // NW)
UNROLL = 4
N_ROUNDS = N_INT // UNROLL


def _sc_body(bx_hbm, by_hbm, sc_hbm, ax_hbm, ay_hbm, tbl_hbm, wf_hbm,
             tbl_v, bx_v, by_v, sc_v, ax_v, ay_v, out_v):
    wid = lax.axis_index("s") * NC + lax.axis_index("c")
    pltpu.sync_copy(tbl_hbm, tbl_v)
    pltpu.sync_copy(ax_hbm, ax_v)
    pltpu.sync_copy(ay_hbm, ay_v)
    axv = ax_v[...]
    ayv = ay_v[...]

    for rr in range(MAX_G_PER_W):
        g = wid + rr * NW

        @pl.when(g < N_GROUPS)
        def _():
            base = g * L
            pltpu.sync_copy(bx_hbm.at[pl.ds(base, L)], bx_v)
            pltpu.sync_copy(by_hbm.at[pl.ds(base, L)], by_v)
            pltpu.sync_copy(sc_hbm.at[pl.ds(base, L)], sc_v)
            bx = bx_v[...]
            by = by_v[...]
            bxu = bx * float(UNROLL)
            byu = by * float(UNROLL)

            def gather_at(xf, yf):
                xi = xf.astype(jnp.int32)
                row = yf.astype(jnp.int32) & 15
                return plsc.load_gather(tbl_v, [(row << 4) + xi])

            zero = jnp.zeros((L,), jnp.float32)
            xfs = tuple(axv + bx * float(k) for k in range(UNROLL))
            yfs = tuple(ayv + by * float(k) for k in range(UNROLL))
            accs = (zero,) * UNROLL

            def body(_, carry):
                xfs, yfs, accs = carry
                nxf, nyf, nacc = [], [], []
                for k in range(UNROLL):
                    nacc.append(accs[k] + gather_at(xfs[k], yfs[k]))
                    nxf.append(xfs[k] + bxu)
                    nyf.append(yfs[k] + byu)
                return tuple(nxf), tuple(nyf), tuple(nacc)

            _, _, accs = lax.fori_loop(0, N_ROUNDS, body, (xfs, yfs, accs))
            acc = functools.reduce(jnp.add, accs)

            v_first = gather_at(axv, ayv)
            v_last = gather_at(axv + bx * float(N_INT - 1),
                               ayv + by * float(N_INT - 1))
            out_v[...] = sc_v[...] * (2.0 * acc - v_first - v_last)
            pltpu.sync_copy(out_v, wf_hbm.at[pl.ds(base, L)])


@functools.cache
def _sc_integrate():
    return pl.kernel(
        _sc_body,
        out_type=jax.ShapeDtypeStruct((N_POINTS,), jnp.float32),
        mesh=plsc.VectorSubcoreMesh(core_axis_name="c", subcore_axis_name="s"),
        compiler_params=pltpu.CompilerParams(needs_layout_passes=False),
        scratch_types=[
            pltpu.VMEM((256,), jnp.float32),
            pltpu.VMEM((L,), jnp.float32),
            pltpu.VMEM((L,), jnp.float32),
            pltpu.VMEM((L,), jnp.float32),
            pltpu.VMEM((L,), jnp.float32),
            pltpu.VMEM((L,), jnp.float32),
            pltpu.VMEM((L,), jnp.float32),
        ],
    )


def kernel(x, y, SoS):
    thetas = jnp.linspace(0.0, 2.0 * jnp.pi, N_POINTS, dtype=jnp.float32)
    r = jnp.sqrt(x ** 2 + y ** 2)
    phi = jnp.arctan2(x, y)
    t = thetas - phi
    chord = jnp.sqrt(R_BODY ** 2 - (r * jnp.sin(t)) ** 2)
    l_inside = chord + r * jnp.cos(t)
    l_outside = 2.0 * chord * (jnp.cos(phi - thetas) >= 0.0).astype(jnp.float32)
    l = jnp.where(r[0] < R_BODY, l_inside, l_outside)

    ax = jnp.full((L,), (x[0] - X0) / DX + 0.5, dtype=jnp.float32)
    ay = jnp.full((L,), -((y[0] - Y0) / DY + 0.5), dtype=jnp.float32)
    bx = -(l * jnp.sin(thetas)) / jnp.float32(DX * (N_INT - 1))
    by = (l * jnp.cos(thetas)) / jnp.float32(DY * (N_INT - 1))
    scale = l / jnp.float32(2 * (N_INT - 1))
    tbl = (1.0 - V0 / SoS).astype(jnp.float32).reshape(-1)

    wf = _sc_integrate()(bx, by, scale, ax, ay, tbl)
    return thetas, wf


# speedup vs baseline: 804.6377x; 1.1530x over previous
import functools

import jax
import jax.numpy as jnp
from jax import lax
from jax.experimental import pallas as pl
from jax.experimental.pallas import tpu as pltpu
from jax.experimental.pallas import tpu_sc as plsc

N_POINTS = 3600
N_INT = 2000
R_BODY = 10.0
V0 = 1540.0
X0 = -12.0
DX = 1.6
Y0 = -12.0
DY = 1.6

L = 16
NC = 2
NS = 16
NW = NC * NS
N_GROUPS = N_POINTS // L
N_GROUPS_PAD = N_GROUPS + 1
UNROLL = 8
N_ROUNDS = N_INT // UNROLL
G_MAX = 8


def _sc_body(params_hbm, consts_hbm, wf_hbm, consts_v, params_v, out_v):
    wid = lax.axis_index("s") * NC + lax.axis_index("c")
    start_g = 7 * wid + jnp.minimum(wid, 1)
    pltpu.sync_copy(consts_hbm, consts_v)
    pltpu.sync_copy(params_hbm.at[pl.ds(start_g * (3 * L), G_MAX * 3 * L)],
                    params_v)
    tbl_v = consts_v.at[pl.ds(0, 256)]
    axv = consts_v[pl.ds(256, L)]
    ayv = consts_v[pl.ds(256 + L, L)]

    for k in range(G_MAX):
        def run_group(k=k):
            off = k * 3 * L
            bx = params_v[pl.ds(off, L)]
            by = params_v[pl.ds(off + L, L)]
            sc = params_v[pl.ds(off + 2 * L, L)]
            bxu = bx * float(UNROLL)
            byu = by * float(UNROLL)

            def gather_at(xf, yf):
                xi = xf.astype(jnp.int32)
                row = yf.astype(jnp.int32) & 15
                return plsc.load_gather(tbl_v, [(row << 4) + xi])

            zero = jnp.zeros((L,), jnp.float32)
            xfs = tuple(axv + bx * float(i) for i in range(UNROLL))
            yfs = tuple(ayv + by * float(i) for i in range(UNROLL))
            accs = (zero,) * UNROLL

            def body(_, carry):
                xfs, yfs, accs = carry
                nxf, nyf, nacc = [], [], []
                for i in range(UNROLL):
                    nacc.append(accs[i] + gather_at(xfs[i], yfs[i]))
                    nxf.append(xfs[i] + bxu)
                    nyf.append(yfs[i] + byu)
                return tuple(nxf), tuple(nyf), tuple(nacc)

            _, _, accs = lax.fori_loop(0, N_ROUNDS, body, (xfs, yfs, accs))
            acc = functools.reduce(jnp.add, accs)

            v_first = gather_at(axv, ayv)
            v_last = gather_at(axv + bx * float(N_INT - 1),
                               ayv + by * float(N_INT - 1))
            out_v[pl.ds(k * L, L)] = sc * (2.0 * acc - v_first - v_last)

        if k < 7:
            run_group()
        else:
            pl.when(wid == 0)(run_group)

    @pl.when(wid == 0)
    def _():
        pltpu.sync_copy(out_v, wf_hbm.at[pl.ds(0, G_MAX * L)])

    @pl.when(wid > 0)
    def _():
        pltpu.sync_copy(out_v.at[pl.ds(0, 7 * L)],
                        wf_hbm.at[pl.ds(G_MAX * L + (wid - 1) * 7 * L, 7 * L)])


@functools.cache
def _sc_integrate():
    return pl.kernel(
        _sc_body,
        out_type=jax.ShapeDtypeStruct((N_POINTS,), jnp.float32),
        mesh=plsc.VectorSubcoreMesh(core_axis_name="c", subcore_axis_name="s"),
        compiler_params=pltpu.CompilerParams(needs_layout_passes=False),
        scratch_types=[
            pltpu.VMEM((256 + 2 * L,), jnp.float32),
            pltpu.VMEM((G_MAX * 3 * L,), jnp.float32),
            pltpu.VMEM((G_MAX * L,), jnp.float32),
        ],
    )


def kernel(x, y, SoS):
    thetas = jnp.linspace(0.0, 2.0 * jnp.pi, N_POINTS, dtype=jnp.float32)
    r = jnp.sqrt(x ** 2 + y ** 2)
    phi = jnp.arctan2(x, y)
    t = thetas - phi
    chord = jnp.sqrt(R_BODY ** 2 - (r * jnp.sin(t)) ** 2)
    l_inside = chord + r * jnp.cos(t)
    l_outside = 2.0 * chord * (jnp.cos(phi - thetas) >= 0.0).astype(jnp.float32)
    l = jnp.where(r[0] < R_BODY, l_inside, l_outside)

    ax = jnp.full((L,), (x[0] - X0) / DX + 0.5, dtype=jnp.float32)
    ay = jnp.full((L,), -((y[0] - Y0) / DY + 0.5), dtype=jnp.float32)
    bx = -(l * jnp.sin(thetas)) / jnp.float32(DX * (N_INT - 1))
    by = (l * jnp.cos(thetas)) / jnp.float32(DY * (N_INT - 1))
    scale = l / jnp.float32(2 * (N_INT - 1))
    tbl = (1.0 - V0 / SoS).astype(jnp.float32).reshape(-1)

    pad = N_GROUPS_PAD * L - N_POINTS
    params = jnp.stack([
        jnp.pad(bx, (0, pad)).reshape(N_GROUPS_PAD, L),
        jnp.pad(by, (0, pad)).reshape(N_GROUPS_PAD, L),
        jnp.pad(scale, (0, pad)).reshape(N_GROUPS_PAD, L),
    ], axis=1).reshape(-1)
    consts = jnp.concatenate([tbl, ax, ay])

    wf = _sc_integrate()(params, consts)
    return thetas, wf


# speedup vs baseline: 1900.5491x; 2.3620x over previous
import functools

import jax
import jax.numpy as jnp
from jax import lax
from jax.experimental import pallas as pl
from jax.experimental.pallas import tpu as pltpu
from jax.experimental.pallas import tpu_sc as plsc

N_POINTS = 3600
N_INT = 2000
R_BODY = 10.0
V0 = 1540.0
X0 = -12.0
DX = 1.6
Y0 = -12.0
DY = 1.6

L = 16
NC = 2
NS = 16
NW = NC * NS
N_GROUPS = N_POINTS // L
N_GROUPS_PAD = N_GROUPS + 1
G_MAX = 8
EPS = 1e-12

O_AX = 4096
O_AY = 4112
O_CX = 4128
O_CY = 4400
O_TBL = 4672
CONSTS_LEN = 4928


def _sc_body(params_hbm, consts_hbm, wf_hbm, consts_v, params_v, yint_v, out_v):
    wid = lax.axis_index("s") * NC + lax.axis_index("c")
    start_g = 7 * wid + jnp.minimum(wid, 1)
    pltpu.sync_copy(consts_hbm, consts_v)
    pltpu.sync_copy(params_hbm.at[pl.ds(start_g * (3 * L), G_MAX * 3 * L)],
                    params_v)
    tbl_v = consts_v.at[pl.ds(O_TBL, 256)]
    axv = consts_v[pl.ds(O_AX, L)]
    ayv = consts_v[pl.ds(O_AY, L)]
    iota = jax.lax.iota(jnp.int32, L)
    n_w = jnp.where(wid == 0, G_MAX, 7)

    def gather_at(xf, yf):
        xi = xf.astype(jnp.int32)
        row = yf.astype(jnp.int32) & 15
        return plsc.load_gather(tbl_v, [(row << 4) + xi])

    def run_group(i, _):
        off = i * (3 * L)
        bx = plsc.load_gather(params_v, [off + iota])
        by = plsc.load_gather(params_v, [off + L + iota])
        sc = plsc.load_gather(params_v, [off + 2 * L + iota])

        def boundaries(b, c_base):
            bp = jnp.where(b >= 0, jnp.maximum(b, EPS), jnp.minimum(b, -EPS))
            inv = 1.0 / bp
            bpos = bp > 0.0

            def q_at(m):
                t = consts_v[pl.ds(c_base + m * L, L)] * inv
                t = jnp.clip(t, -1.0, float(N_INT + 1))
                ti = t.astype(jnp.int32)
                tf = ti.astype(jnp.float32)
                up = jnp.where(jnp.where(bpos, t > tf, t >= tf), 1, 0)
                return jnp.clip(ti + up, 0, N_INT)

            q_prev = q_at(0)
            los, hips = [], []
            for m in range(1, 17):
                q_cur = q_at(m)
                los.append(jnp.minimum(q_prev, q_cur))
                hips.append(jnp.maximum(q_prev, q_cur))
                q_prev = q_cur
            return los, hips

        ylos, yhips = boundaries(by, O_CY)
        for k in range(16):
            yint_v[pl.ds(k * L, L)] = ylos[k]
            yint_v[pl.ds((16 + k) * L, L)] = yhips[k]
        xlos, xhips = boundaries(bx, O_CX)

        acc = jnp.zeros((L,), jnp.float32)
        for k in range(16):
            ylo = yint_v[pl.ds(k * L, L)]
            yhip = yint_v[pl.ds((16 + k) * L, L)]
            row = (k + 1) & 15
            for m in range(16):
                cnt = jnp.minimum(xhips[m], yhip) - jnp.maximum(xlos[m], ylo)
                cnt = jnp.maximum(cnt, 0).astype(jnp.float32)
                tv = consts_v[pl.ds((row * 16 + m) * L, L)]
                acc = acc + cnt * tv

        v_first = gather_at(axv, ayv)
        v_last = gather_at(axv + bx * float(N_INT - 1),
                           ayv + by * float(N_INT - 1))
        wf = sc * (2.0 * acc - v_first - v_last)
        plsc.store_scatter(out_v, [i * L + iota], wf)
        return 0

    lax.fori_loop(0, n_w, run_group, 0)

    @pl.when(wid == 0)
    def _():
        pltpu.sync_copy(out_v, wf_hbm.at[pl.ds(0, G_MAX * L)])

    @pl.when(wid > 0)
    def _():
        pltpu.sync_copy(out_v.at[pl.ds(0, 7 * L)],
                        wf_hbm.at[pl.ds(G_MAX * L + (wid - 1) * 7 * L, 7 * L)])


@functools.cache
def _sc_integrate():
    return pl.kernel(
        _sc_body,
        out_type=jax.ShapeDtypeStruct((N_POINTS,), jnp.float32),
        mesh=plsc.VectorSubcoreMesh(core_axis_name="c", subcore_axis_name="s"),
        compiler_params=pltpu.CompilerParams(needs_layout_passes=False),
        scratch_types=[
            pltpu.VMEM((CONSTS_LEN,), jnp.float32),
            pltpu.VMEM((G_MAX * 3 * L,), jnp.float32),
            pltpu.VMEM((2 * 16 * L,), jnp.int32),
            pltpu.VMEM((G_MAX * L,), jnp.float32),
        ],
    )


def kernel(x, y, SoS):
    thetas = jnp.linspace(0.0, 2.0 * jnp.pi, N_POINTS, dtype=jnp.float32)
    r = jnp.sqrt(x ** 2 + y ** 2)
    phi = jnp.arctan2(x, y)
    t = thetas - phi
    chord = jnp.sqrt(R_BODY ** 2 - (r * jnp.sin(t)) ** 2)
    l_inside = chord + r * jnp.cos(t)
    l_outside = 2.0 * chord * (jnp.cos(phi - thetas) >= 0.0).astype(jnp.float32)
    l = jnp.where(r[0] < R_BODY, l_inside, l_outside)

    ax = (x[0] - X0) / DX + 0.5
    ay = -((y[0] - Y0) / DY + 0.5)
    bx = -(l * jnp.sin(thetas)) / jnp.float32(DX * (N_INT - 1))
    by = (l * jnp.cos(thetas)) / jnp.float32(DY * (N_INT - 1))
    scale = l / jnp.float32(2 * (N_INT - 1))
    tbl = (1.0 - V0 / SoS).astype(jnp.float32).reshape(-1)

    pad = N_GROUPS_PAD * L - N_POINTS
    params = jnp.stack([
        jnp.pad(bx, (0, pad)).reshape(N_GROUPS_PAD, L),
        jnp.pad(by, (0, pad)).reshape(N_GROUPS_PAD, L),
        jnp.pad(scale, (0, pad)).reshape(N_GROUPS_PAD, L),
    ], axis=1).reshape(-1)

    marange = jnp.arange(17, dtype=jnp.float32)
    consts = jnp.concatenate([
        jnp.repeat(tbl, L),
        jnp.full((L,), ax, dtype=jnp.float32),
        jnp.full((L,), ay, dtype=jnp.float32),
        jnp.repeat((marange - ax).astype(jnp.float32), L),
        jnp.repeat((marange - 16.0 - ay).astype(jnp.float32), L),
        tbl,
    ])

    wf = _sc_integrate()(params, consts)
    return thetas, wf


# speedup vs baseline: 1971.5239x; 1.0373x over previous
import functools

import jax
import jax.numpy as jnp
from jax import lax
from jax.experimental import pallas as pl
from jax.experimental.pallas import tpu as pltpu
from jax.experimental.pallas import tpu_sc as plsc

N_POINTS = 3600
N_INT = 2000
R_BODY = 10.0
V0 = 1540.0
X0 = -12.0
DX = 1.6
Y0 = -12.0
DY = 1.6

L = 16
NC = 2
NS = 16
NW = NC * NS
N_GROUPS = N_POINTS // L
N_GROUPS_PAD = N_GROUPS + 1
G_MAX = 8
EPS = 1e-12

O_AX = 4096
O_AY = 4112
O_CX = 4128
O_CY = 4400
O_TBL = 4672
CONSTS_LEN = 4928
PARAMS_LEN = (N_GROUPS + 1) * 3 * 16


def _sc_body(in_hbm, wf_hbm, consts_v, params_v, yint_v, out_v, sem1, sem2):
    wid = lax.axis_index("s") * NC + lax.axis_index("c")
    start_g = 7 * wid + jnp.minimum(wid, 1)
    cp1 = pltpu.async_copy(in_hbm.at[pl.ds(PARAMS_LEN, CONSTS_LEN)], consts_v,
                           sem1)
    cp2 = pltpu.async_copy(
        in_hbm.at[pl.ds(start_g * (3 * L), G_MAX * 3 * L)], params_v, sem2)
    cp1.wait()
    cp2.wait()
    tbl_v = consts_v.at[pl.ds(O_TBL, 256)]
    axv = consts_v[pl.ds(O_AX, L)]
    ayv = consts_v[pl.ds(O_AY, L)]
    iota = jax.lax.iota(jnp.int32, L)
    n_w = jnp.where(wid == 0, G_MAX, 7)

    def gather_at(xf, yf):
        xi = xf.astype(jnp.int32)
        row = yf.astype(jnp.int32) & 15
        return plsc.load_gather(tbl_v, [(row << 4) + xi])

    def run_group(i, _):
        off = i * (3 * L)
        bx = plsc.load_gather(params_v, [off + iota])
        by = plsc.load_gather(params_v, [off + L + iota])
        sc = plsc.load_gather(params_v, [off + 2 * L + iota])

        def boundaries(b, c_base):
            bp = jnp.where(b >= 0, jnp.maximum(b, EPS), jnp.minimum(b, -EPS))
            inv = 1.0 / bp
            bpos = bp > 0.0

            def q_at(m):
                t = consts_v[pl.ds(c_base + m * L, L)] * inv
                t = jnp.clip(t, -1.0, float(N_INT + 1))
                ti = t.astype(jnp.int32)
                tf = ti.astype(jnp.float32)
                up = jnp.where(jnp.where(bpos, t > tf, t >= tf), 1, 0)
                return jnp.clip(ti + up, 0, N_INT)

            q_prev = q_at(0)
            los, hips = [], []
            for m in range(1, 17):
                q_cur = q_at(m)
                los.append(jnp.minimum(q_prev, q_cur))
                hips.append(jnp.maximum(q_prev, q_cur))
                q_prev = q_cur
            return los, hips

        ylos, yhips = boundaries(by, O_CY)
        for k in range(16):
            yint_v[pl.ds(k * L, L)] = ylos[k]
            yint_v[pl.ds((16 + k) * L, L)] = yhips[k]
        xlos, xhips = boundaries(bx, O_CX)

        acc = jnp.zeros((L,), jnp.float32)
        for k in range(16):
            ylo = yint_v[pl.ds(k * L, L)]
            yhip = yint_v[pl.ds((16 + k) * L, L)]
            row = (k + 1) & 15
            for m in range(16):
                cnt = jnp.minimum(xhips[m], yhip) - jnp.maximum(xlos[m], ylo)
                cnt = jnp.maximum(cnt, 0).astype(jnp.float32)
                tv = consts_v[pl.ds((row * 16 + m) * L, L)]
                acc = acc + cnt * tv

        v_first = gather_at(axv, ayv)
        v_last = gather_at(axv + bx * float(N_INT - 1),
                           ayv + by * float(N_INT - 1))
        wf = sc * (2.0 * acc - v_first - v_last)
        plsc.store_scatter(out_v, [i * L + iota], wf)
        return 0

    lax.fori_loop(0, n_w, run_group, 0)

    @pl.when(wid == 0)
    def _():
        pltpu.sync_copy(out_v, wf_hbm.at[pl.ds(0, G_MAX * L)])

    @pl.when(wid > 0)
    def _():
        pltpu.sync_copy(out_v.at[pl.ds(0, 7 * L)],
                        wf_hbm.at[pl.ds(G_MAX * L + (wid - 1) * 7 * L, 7 * L)])


@functools.cache
def _sc_integrate():
    return pl.kernel(
        _sc_body,
        out_type=jax.ShapeDtypeStruct((N_POINTS,), jnp.float32),
        mesh=plsc.VectorSubcoreMesh(core_axis_name="c", subcore_axis_name="s"),
        compiler_params=pltpu.CompilerParams(needs_layout_passes=False),
        scratch_types=[
            pltpu.VMEM((CONSTS_LEN,), jnp.float32),
            pltpu.VMEM((G_MAX * 3 * L,), jnp.float32),
            pltpu.VMEM((2 * 16 * L,), jnp.int32),
            pltpu.VMEM((G_MAX * L,), jnp.float32),
            pltpu.SemaphoreType.DMA,
            pltpu.SemaphoreType.DMA,
        ],
    )


def kernel(x, y, SoS):
    thetas = jnp.linspace(0.0, 2.0 * jnp.pi, N_POINTS, dtype=jnp.float32)
    r = jnp.sqrt(x ** 2 + y ** 2)
    phi = jnp.arctan2(x, y)
    t = thetas - phi
    chord = jnp.sqrt(R_BODY ** 2 - (r * jnp.sin(t)) ** 2)
    l_inside = chord + r * jnp.cos(t)
    l_outside = 2.0 * chord * (jnp.cos(phi - thetas) >= 0.0).astype(jnp.float32)
    l = jnp.where(r[0] < R_BODY, l_inside, l_outside)

    ax = (x[0] - X0) / DX + 0.5
    ay = -((y[0] - Y0) / DY + 0.5)
    bx = -(l * jnp.sin(thetas)) / jnp.float32(DX * (N_INT - 1))
    by = (l * jnp.cos(thetas)) / jnp.float32(DY * (N_INT - 1))
    scale = l / jnp.float32(2 * (N_INT - 1))
    tbl = (1.0 - V0 / SoS).astype(jnp.float32).reshape(-1)

    pad = N_GROUPS_PAD * L - N_POINTS
    params = jnp.stack([
        jnp.pad(bx, (0, pad)).reshape(N_GROUPS_PAD, L),
        jnp.pad(by, (0, pad)).reshape(N_GROUPS_PAD, L),
        jnp.pad(scale, (0, pad)).reshape(N_GROUPS_PAD, L),
    ], axis=1).reshape(-1)

    marange = jnp.arange(17, dtype=jnp.float32)
    consts = jnp.concatenate([
        jnp.repeat(tbl, L),
        jnp.full((L,), ax, dtype=jnp.float32),
        jnp.full((L,), ay, dtype=jnp.float32),
        jnp.repeat((marange - ax).astype(jnp.float32), L),
        jnp.repeat((marange - 16.0 - ay).astype(jnp.float32), L),
        tbl,
    ])

    wf = _sc_integrate()(jnp.concatenate([params, consts]))
    return thetas, wf
